# baseline (speedup 1.0000x reference)
import jax
import jax.numpy as jnp
from jax import lax
from jax.experimental import pallas as pl
from jax.experimental.pallas import tpu as pltpu
from jax.experimental.pallas import tpu_sc as plsc

BATCH = 16384
FACTOR = 64
NUM_WORKERS = 32
ROWS_PER_WORKER = BATCH // NUM_WORKERS
CHUNK = 128
NCHUNKS = ROWS_PER_WORKER // CHUNK
PAIRW = 2 * FACTOR


def _clmf_body(user_hbm, item_hbm, ut_hbm, it_hbm, wb_hbm, out_hbm,
               idx_u, idx_i, pair_u, pair_i,
               eu0, eu1, ei0, ei1, out_v, wb_v,
               sem_u0, sem_u1, sem_i0, sem_i1):
    wid = lax.axis_index("s") * 2 + lax.axis_index("c")
    base = wid * ROWS_PER_WORKER

    pltpu.sync_copy(user_hbm.at[pl.ds(base, ROWS_PER_WORKER)], idx_u)
    pltpu.sync_copy(item_hbm.at[pl.ds(base, ROWS_PER_WORKER)], idx_i)
    pltpu.sync_copy(wb_hbm, wb_v)

    for v in range(ROWS_PER_WORKER // 16):
        sl = pl.ds(16 * v, 16)
        pair_u[sl] = lax.shift_right_logical(idx_u[sl], 1)
        pair_i[sl] = lax.shift_right_logical(idx_i[sl], 1)

    lane = lax.iota(jnp.int32, 16)
    w_vecs = [wb_v[pl.ds(16 * q, 16)] for q in range(FACTOR // 16)]
    bias_vec = wb_v[pl.ds(FACTOR, 16)]

    def fire(c, eu_buf, ei_buf, sem_u, sem_i):
        pltpu.async_copy(ut_hbm.at[pair_u.at[pl.ds(CHUNK * c, CHUNK)]],
                         eu_buf, sem_u)
        pltpu.async_copy(it_hbm.at[pair_i.at[pl.ds(CHUNK * c, CHUNK)]],
                         ei_buf, sem_i)

    def wait(eu_buf, ei_buf, sem_u, sem_i):
        pltpu.make_async_copy(ut_hbm.at[pair_u.at[pl.ds(0, CHUNK)]],
                              eu_buf, sem_u).wait()
        pltpu.make_async_copy(it_hbm.at[pair_i.at[pl.ds(0, CHUNK)]],
                              ei_buf, sem_i).wait()

    def compute(c, eu_buf, ei_buf):
        def block(b, carry):
            off = CHUNK * c + 16 * b
            iu = idx_u[pl.ds(off, 16)]
            ii = idx_i[pl.ds(off, 16)]
            cu = lax.bitwise_and(iu, 1) * FACTOR
            ci = lax.bitwise_and(ii, 1) * FACTOR
            rows = 16 * b + lane
            acc = bias_vec
            for f in range(FACTOR):
                eu_f = plsc.load_gather(eu_buf, [rows, cu + f])
                ei_f = plsc.load_gather(ei_buf, [rows, ci + f])
                acc = acc + eu_f * ei_f * w_vecs[f // 16][f % 16]
            out_v[pl.ds(off, 16)] = acc
            return carry

        lax.fori_loop(0, CHUNK // 16, block, 0)

    fire(0, eu0, ei0, sem_u0, sem_i0)
    for c in range(NCHUNKS):
        bufs = (eu0, ei0, sem_u0, sem_i0) if c % 2 == 0 else \
               (eu1, ei1, sem_u1, sem_i1)
        if c + 1 < NCHUNKS:
            nxt = (eu1, ei1, sem_u1, sem_i1) if c % 2 == 0 else \
                  (eu0, ei0, sem_u0, sem_i0)
            fire(c + 1, *nxt)
        wait(*bufs)
        compute(c, bufs[0], bufs[1])

    pltpu.sync_copy(out_v, out_hbm.at[pl.ds(base, ROWS_PER_WORKER)])


@jax.jit
def _clmf_call(user, item, ut2, it2, wb):
    mesh = plsc.VectorSubcoreMesh(core_axis_name="c", subcore_axis_name="s")
    kern = pl.kernel(
        _clmf_body,
        out_type=jax.ShapeDtypeStruct((BATCH,), jnp.float32),
        mesh=mesh,
        compiler_params=pltpu.CompilerParams(needs_layout_passes=False),
        scratch_types=[
            pltpu.VMEM((ROWS_PER_WORKER,), jnp.int32),
            pltpu.VMEM((ROWS_PER_WORKER,), jnp.int32),
            pltpu.VMEM((ROWS_PER_WORKER,), jnp.int32),
            pltpu.VMEM((ROWS_PER_WORKER,), jnp.int32),
            pltpu.VMEM((CHUNK, PAIRW), jnp.float32),
            pltpu.VMEM((CHUNK, PAIRW), jnp.float32),
            pltpu.VMEM((CHUNK, PAIRW), jnp.float32),
            pltpu.VMEM((CHUNK, PAIRW), jnp.float32),
            pltpu.VMEM((ROWS_PER_WORKER,), jnp.float32),
            pltpu.VMEM((FACTOR + 16,), jnp.float32),
            pltpu.SemaphoreType.DMA,
            pltpu.SemaphoreType.DMA,
            pltpu.SemaphoreType.DMA,
            pltpu.SemaphoreType.DMA,
        ],
    )
    return kern(user, item, ut2, it2, wb)


def kernel(user, item, embed_user_w, embed_item_w, predict_w, predict_b):
    user = user.astype(jnp.int32)
    item = item.astype(jnp.int32)
    ut2 = embed_user_w.reshape(-1, PAIRW)
    it2 = embed_item_w.reshape(-1, PAIRW)
    w = predict_w.reshape(FACTOR).astype(jnp.float32)
    b = jnp.broadcast_to(predict_b.astype(jnp.float32), (16,))
    wb = jnp.concatenate([w, b])
    return _clmf_call(user, item, ut2, it2, wb)


# speedup vs baseline: 1.5479x; 1.5479x over previous
import jax
import jax.numpy as jnp
from jax import lax
from jax.experimental import pallas as pl
from jax.experimental.pallas import tpu as pltpu
from jax.experimental.pallas import tpu_sc as plsc

BATCH = 16384
FACTOR = 64
NUM_WORKERS = 32
ROWS_PER_WORKER = BATCH // NUM_WORKERS
CHUNK = 64
NCHUNKS = ROWS_PER_WORKER // CHUNK


def _clmf_body(user_hbm, item_hbm, ut_hbm, it_hbm, wb_hbm, out_hbm,
               idx_u, idx_i,
               eu0, eu1, ei0, ei1, out_v, wb_v,
               sem_u0, sem_u1, sem_i0, sem_i1):
    wid = lax.axis_index("s") * 2 + lax.axis_index("c")
    base = wid * ROWS_PER_WORKER

    pltpu.sync_copy(user_hbm.at[pl.ds(base, ROWS_PER_WORKER)], idx_u)
    pltpu.sync_copy(item_hbm.at[pl.ds(base, ROWS_PER_WORKER)], idx_i)
    pltpu.sync_copy(wb_hbm, wb_v)

    lane = lax.iota(jnp.int32, 16)
    w_vecs = [wb_v[pl.ds(16 * q, 16)] for q in range(FACTOR // 16)]
    bias_vec = wb_v[pl.ds(FACTOR, 16)]

    def fire(c, eu_buf, ei_buf, sem_u, sem_i):
        def group(g, carry):
            vu = idx_u[pl.ds(CHUNK * c + 16 * g, 16)]
            vi = idx_i[pl.ds(CHUNK * c + 16 * g, 16)]
            for j in range(16):
                pltpu.async_copy(ut_hbm.at[pl.ds(vu[j], 1)],
                                 eu_buf.at[pl.ds(16 * g + j, 1)], sem_u)
                pltpu.async_copy(it_hbm.at[pl.ds(vi[j], 1)],
                                 ei_buf.at[pl.ds(16 * g + j, 1)], sem_i)
            return carry

        lax.fori_loop(0, CHUNK // 16, group, 0)

    def wait(eu_buf, ei_buf, sem_u, sem_i):
        pltpu.make_async_copy(ut_hbm.at[pl.ds(0, CHUNK)],
                              eu_buf, sem_u).wait()
        pltpu.make_async_copy(it_hbm.at[pl.ds(0, CHUNK)],
                              ei_buf, sem_i).wait()

    def compute(c, eu_buf, ei_buf):
        def block(b, carry):
            off = CHUNK * c + 16 * b
            rows = 16 * b + lane
            acc = bias_vec
            for f in range(FACTOR):
                col = jnp.full((16,), f, dtype=jnp.int32)
                eu_f = plsc.load_gather(eu_buf, [rows, col])
                ei_f = plsc.load_gather(ei_buf, [rows, col])
                acc = acc + eu_f * ei_f * w_vecs[f // 16][f % 16]
            out_v[pl.ds(off, 16)] = acc
            return carry

        lax.fori_loop(0, CHUNK // 16, block, 0)

    fire(0, eu0, ei0, sem_u0, sem_i0)
    for c in range(NCHUNKS):
        bufs = (eu0, ei0, sem_u0, sem_i0) if c % 2 == 0 else \
               (eu1, ei1, sem_u1, sem_i1)
        if c + 1 < NCHUNKS:
            nxt = (eu1, ei1, sem_u1, sem_i1) if c % 2 == 0 else \
                  (eu0, ei0, sem_u0, sem_i0)
            fire(c + 1, *nxt)
        wait(*bufs)
        compute(c, bufs[0], bufs[1])

    pltpu.sync_copy(out_v, out_hbm.at[pl.ds(base, ROWS_PER_WORKER)])


@jax.jit
def _clmf_call(user, item, ut, it, wb):
    mesh = plsc.VectorSubcoreMesh(core_axis_name="c", subcore_axis_name="s")
    kern = pl.kernel(
        _clmf_body,
        out_type=jax.ShapeDtypeStruct((BATCH,), jnp.float32),
        mesh=mesh,
        compiler_params=pltpu.CompilerParams(needs_layout_passes=False,
                                             use_tc_tiling_on_sc=True),
        scratch_types=[
            pltpu.VMEM((ROWS_PER_WORKER,), jnp.int32),
            pltpu.VMEM((ROWS_PER_WORKER,), jnp.int32),
            pltpu.VMEM((CHUNK, FACTOR), jnp.float32),
            pltpu.VMEM((CHUNK, FACTOR), jnp.float32),
            pltpu.VMEM((CHUNK, FACTOR), jnp.float32),
            pltpu.VMEM((CHUNK, FACTOR), jnp.float32),
            pltpu.VMEM((ROWS_PER_WORKER,), jnp.float32),
            pltpu.VMEM((FACTOR + 16,), jnp.float32),
            pltpu.SemaphoreType.DMA,
            pltpu.SemaphoreType.DMA,
            pltpu.SemaphoreType.DMA,
            pltpu.SemaphoreType.DMA,
        ],
    )
    return kern(user, item, ut, it, wb)


def kernel(user, item, embed_user_w, embed_item_w, predict_w, predict_b):
    user = user.astype(jnp.int32)
    item = item.astype(jnp.int32)
    w = predict_w.reshape(FACTOR).astype(jnp.float32)
    b = jnp.broadcast_to(predict_b.astype(jnp.float32), (16,))
    wb = jnp.concatenate([w, b])
    return _clmf_call(user, item, embed_user_w, embed_item_w, wb)
